# baseline (speedup 1.0000x reference)
import functools

import jax
import jax.numpy as jnp
from jax import lax
from jax.experimental import pallas as pl
from jax.experimental.pallas import tpu as pltpu
from jax.experimental.pallas import tpu_sc as plsc

_NC = 2
_NS = 16
_LANES = 16


def _s1_body(fea_ref, c_ref, s_ref):
    i = pl.program_id(0)
    f = fea_ref[...]

    @pl.when(i == 0)
    def _():
        c_ref[...] = jnp.zeros_like(c_ref)
        s_ref[...] = jnp.zeros_like(s_ref)

    c_ref[...] += lax.dot_general(f, f, (((0,), (0,)), ((), ())),
                                  preferred_element_type=jnp.float32)
    s_ref[...] += jnp.sum(f, axis=0, keepdims=True)


def _make_s2(n_rows, d_half):
    def body(fea_ref, wl_ref, bl_ref, c_ref, s_ref, g_ref, b_ref, wg_ref,
             d0_ref, d1_ref, xz_ref):
        inv_n = 1.0 / n_rows
        wl = wl_ref[...]
        x = jnp.dot(fea_ref[...], wl, preferred_element_type=jnp.float32)
        x = x + bl_ref[...]
        mean_fw = jnp.dot(s_ref[...] * inv_n, wl,
                          preferred_element_type=jnp.float32)
        cw = jnp.dot(c_ref[...], wl, preferred_element_type=jnp.float32)
        ex2 = jnp.sum(wl * cw, axis=0, keepdims=True) * inv_n
        var = ex2 - mean_fw * mean_fw
        mean = mean_fw + bl_ref[...]
        rstd = lax.rsqrt(var + 1e-5)
        xn = (x - mean) * rstd * g_ref[...] + b_ref[...]
        xr = jnp.where(xn >= 0, xn, 0.01 * xn)
        xw = jnp.dot(xr, wg_ref[...], preferred_element_type=jnp.float32)
        deg = d0_ref[...] + d1_ref[...] + 1.0
        dinv = lax.rsqrt(deg)
        xz = xw * dinv
        xz_ref[0] = xz[:, :d_half]
        xz_ref[1] = xz[:, d_half:]
    return body


def _make_s4(d_half):
    def body(al_ref, ar_ref, xl_ref, xr_ref, d0_ref, d1_ref, bg_ref, out_ref):
        deg = d0_ref[...] + d1_ref[...] + 1.0
        dinv = lax.rsqrt(deg)
        out_ref[:, :d_half] = dinv * (al_ref[...] + xl_ref[...]) + bg_ref[:, :d_half]
        out_ref[:, d_half:] = dinv * (ar_ref[...] + xr_ref[...]) + bg_ref[:, d_half:]
    return body


def _make_deg(np_pad, ep32, ch32):
    rt = np_pad // _NS

    def body(cols3, w_hbm, degp, cbuf, wbuf, dbuf, deg_sp):
        c = lax.axis_index("c")
        s = lax.axis_index("s")
        t = c * _NS + s

        def zstep(i, carry):
            dbuf[pl.ds(i * _LANES, _LANES)] = jnp.zeros((_LANES,), jnp.float32)
            return carry

        lax.fori_loop(0, rt // _LANES, zstep, 0)
        pltpu.sync_copy(dbuf, deg_sp.at[pl.ds(s * rt, rt)])
        pltpu.sync_copy(cols3.at[t], cbuf)
        pltpu.sync_copy(w_hbm.at[pl.ds(t * ep32, ep32)], wbuf)
        plsc.subcore_barrier()

        def chunk(g, carry):
            pltpu.sync_copy(wbuf.at[pl.ds(g * 128, 128)],
                            deg_sp.at[cbuf.at[g]], add=True)
            return carry

        lax.fori_loop(0, ch32, chunk, 0)
        plsc.subcore_barrier()
        pltpu.sync_copy(deg_sp.at[pl.ds(s * rt, rt)], dbuf)
        pltpu.sync_copy(dbuf, degp.at[pl.ds(c * np_pad + s * rt, rt)])
    return body


_CK = 64
_NBUF = 4


def _make_agg(np_pad, d_half, ep16, ch16):
    rt = np_pad // _NS
    nvec = d_half // _LANES

    def body(rows2, cols_hbm, w_hbm, xzflat, agg3,
             ibufs, cbufs, vbufs, gbufs, acc, sems_g, sems_s, sems_i,
             sems_c):
        c = lax.axis_index("c")
        s = lax.axis_index("s")
        ebase = s * ep16

        def zstep(i, carry):
            for j in range(nvec):
                gbufs[0][i, pl.ds(j * _LANES, _LANES)] = (
                    jnp.zeros((_LANES,), jnp.float32))
            return carry

        lax.fori_loop(0, _CK, zstep, 0)
        for k in range(rt // _CK):
            pltpu.sync_copy(gbufs[0], acc.at[pl.ds(s * rt + k * _CK, _CK)])
        plsc.subcore_barrier()

        def idx_load(g, b):
            pltpu.async_copy(rows2.at[c, pl.ds(ebase + g * _CK, _CK)],
                             ibufs[b], sems_i[b])
            pltpu.async_copy(w_hbm.at[pl.ds(ebase + g * _CK, _CK)],
                             vbufs[b], sems_i[b])

        def idx_wait(g, b):
            pltpu.make_async_copy(rows2.at[c, pl.ds(ebase + g * _CK, _CK)],
                                  ibufs[b], sems_i[b]).wait()
            pltpu.make_async_copy(w_hbm.at[pl.ds(ebase + g * _CK, _CK)],
                                  vbufs[b], sems_i[b]).wait()

        def cols_load(g, b):
            pltpu.async_copy(cols_hbm.at[pl.ds(ebase + g * _CK, _CK)],
                             cbufs[b], sems_c[b])

        def cols_wait(g, b):
            pltpu.make_async_copy(cols_hbm.at[pl.ds(ebase + g * _CK, _CK)],
                                  cbufs[b], sems_c[b]).wait()

        def gather(b):
            pltpu.async_copy(xzflat.at[ibufs[b]], gbufs[b], sems_g[b])

        def gather_wait(b):
            pltpu.make_async_copy(xzflat.at[ibufs[b]], gbufs[b],
                                  sems_g[b]).wait()

        def scale(b):
            gb = gbufs[b]
            vb = vbufs[b]

            @plsc.parallel_loop(0, _CK, step=1, unroll=4)
            def _(e):
                idx = jnp.broadcast_to(e, (_LANES,)).astype(jnp.int32)
                sp = plsc.load_gather(vb, [idx])
                for j in range(nvec):
                    sl = pl.ds(j * _LANES, _LANES)
                    gb[e, sl] = gb[e, sl] * sp

        def scatter(b):
            pltpu.async_copy(gbufs[b], acc.at[cbufs[b]], sems_s[b], add=True)

        def scatter_wait(b):
            pltpu.make_async_copy(gbufs[b], acc.at[cbufs[b]],
                                  sems_s[b]).wait()

        for q in range(_NBUF):
            idx_load(q, q)
            cols_load(q, q)
        for q in range(_NBUF - 1):
            idx_wait(q, q)
            gather(q)

        def quad(p, carry):
            for q in range(_NBUF):
                g = p * _NBUF + q
                b = q
                bp = (q - 1) % _NBUF
                gather_wait(b)
                scale(b)

                @pl.when(g > 0)
                def _():
                    scatter_wait(bp)

                    @pl.when(g + _NBUF - 1 < ch16)
                    def _():
                        cols_load(g + _NBUF - 1, bp)

                @pl.when(g + _NBUF - 1 < ch16)
                def _():
                    idx_wait(g + _NBUF - 1, bp)
                    gather(bp)

                @pl.when(g + _NBUF < ch16)
                def _():
                    idx_load(g + _NBUF, b)

                cols_wait(g, b)
                scatter(b)
            return carry

        lax.fori_loop(0, ch16 // _NBUF, quad, 0)
        scatter_wait(_NBUF - 1)
        plsc.subcore_barrier()
        for k in range(rt // _CK):
            pltpu.sync_copy(acc.at[pl.ds(s * rt + k * _CK, _CK)], gbufs[0])
            pltpu.sync_copy(
                gbufs[0], agg3.at[pl.ds(c * np_pad + s * rt + k * _CK, _CK)])
    return body


def kernel(fea, edges, weights, W_lin, b_lin, gamma, beta, W_gcn, b_gcn):
    N, D = fea.shape
    DH = D // 2
    E = edges.shape[1]

    EPAD = -(-E // (_NC * _NS * 128)) * (_NC * _NS * 128)
    EP16 = EPAD // _NS
    CH16 = EP16 // _CK
    EP32 = EPAD // (_NC * _NS)
    CH32 = EP32 // 128
    NP = -(-N // 2048) * 2048
    R = 1000
    G = N // R

    rows = edges[0].astype(jnp.int32)
    cols = edges[1].astype(jnp.int32)
    pad = EPAD - E
    rows_p = jnp.concatenate([rows, jnp.zeros((pad,), jnp.int32)])
    cols_p = jnp.concatenate([cols, jnp.zeros((pad,), jnp.int32)])
    w_p = jnp.concatenate([weights.astype(jnp.float32),
                           jnp.zeros((pad,), jnp.float32)])
    rows2 = jnp.stack([rows_p, rows_p + N])
    cols3_32 = cols_p.reshape(_NC * _NS, CH32, 128)

    Cmat, fsum = pl.pallas_call(
        _s1_body,
        grid=(G,),
        in_specs=[pl.BlockSpec((R, D), lambda i: (i, 0))],
        out_specs=[pl.BlockSpec((D, D), lambda i: (0, 0)),
                   pl.BlockSpec((1, D), lambda i: (0, 0))],
        out_shape=[jax.ShapeDtypeStruct((D, D), jnp.float32),
                   jax.ShapeDtypeStruct((1, D), jnp.float32)],
    )(fea)

    mesh = plsc.VectorSubcoreMesh(core_axis_name="c", subcore_axis_name="s")
    sc_params = pltpu.CompilerParams(needs_layout_passes=False)
    degp = pl.kernel(
        _make_deg(NP, EP32, CH32),
        out_type=jax.ShapeDtypeStruct((_NC * NP,), jnp.float32),
        mesh=mesh,
        compiler_params=sc_params,
        scratch_types=[
            pltpu.VMEM((CH32, 128), jnp.int32),
            pltpu.VMEM((EP32,), jnp.float32),
            pltpu.VMEM((NP // _NS,), jnp.float32),
            pltpu.VMEM_SHARED((NP,), jnp.float32),
        ],
    )(cols3_32, w_p)

    deg0 = degp[:N].reshape(N, 1)
    deg1 = degp[NP:NP + N].reshape(N, 1)

    xz3 = pl.pallas_call(
        _make_s2(N, DH),
        grid=(G,),
        in_specs=[pl.BlockSpec((R, D), lambda i: (i, 0)),
                  pl.BlockSpec((D, D), lambda i: (0, 0)),
                  pl.BlockSpec((1, D), lambda i: (0, 0)),
                  pl.BlockSpec((D, D), lambda i: (0, 0)),
                  pl.BlockSpec((1, D), lambda i: (0, 0)),
                  pl.BlockSpec((1, D), lambda i: (0, 0)),
                  pl.BlockSpec((1, D), lambda i: (0, 0)),
                  pl.BlockSpec((D, D), lambda i: (0, 0)),
                  pl.BlockSpec((R, 1), lambda i: (i, 0)),
                  pl.BlockSpec((R, 1), lambda i: (i, 0))],
        out_specs=pl.BlockSpec((_NC, R, DH), lambda i: (0, i, 0)),
        out_shape=jax.ShapeDtypeStruct((_NC, N, DH), jnp.float32),
    )(fea, W_lin, b_lin.reshape(1, D), Cmat, fsum, gamma.reshape(1, D),
      beta.reshape(1, D), W_gcn, deg0, deg1)

    xzflat = xz3.reshape(_NC * N, DH)

    agg2 = pl.kernel(
        _make_agg(NP, DH, EP16, CH16),
        out_type=jax.ShapeDtypeStruct((_NC * NP, DH), jnp.float32),
        mesh=mesh,
        compiler_params=sc_params,
        scratch_types=[
            [pltpu.VMEM((_CK,), jnp.int32) for _ in range(_NBUF)],
            [pltpu.VMEM((_CK,), jnp.int32) for _ in range(_NBUF)],
            [pltpu.VMEM((_CK,), jnp.float32) for _ in range(_NBUF)],
            [pltpu.VMEM((_CK, DH), jnp.float32) for _ in range(_NBUF)],
            pltpu.VMEM_SHARED((NP, DH), jnp.float32),
            [pltpu.SemaphoreType.DMA for _ in range(_NBUF)],
            [pltpu.SemaphoreType.DMA for _ in range(_NBUF)],
            [pltpu.SemaphoreType.DMA for _ in range(_NBUF)],
            [pltpu.SemaphoreType.DMA for _ in range(_NBUF)],
        ],
    )(rows2, cols_p, w_p, xzflat)
    agg_l = agg2[:N]
    agg_r = agg2[NP:NP + N]
    xz_l = xz3[0]
    xz_r = xz3[1]

    out = pl.pallas_call(
        _make_s4(DH),
        grid=(G,),
        in_specs=[pl.BlockSpec((R, DH), lambda i: (i, 0)),
                  pl.BlockSpec((R, DH), lambda i: (i, 0)),
                  pl.BlockSpec((R, DH), lambda i: (i, 0)),
                  pl.BlockSpec((R, DH), lambda i: (i, 0)),
                  pl.BlockSpec((R, 1), lambda i: (i, 0)),
                  pl.BlockSpec((R, 1), lambda i: (i, 0)),
                  pl.BlockSpec((1, D), lambda i: (0, 0))],
        out_specs=pl.BlockSpec((R, D), lambda i: (i, 0)),
        out_shape=jax.ShapeDtypeStruct((N, D), jnp.float32),
    )(agg_l, agg_r, xz_l, xz_r, deg0, deg1, b_gcn.reshape(1, D))
    return out


# speedup vs baseline: 1.0678x; 1.0678x over previous
import functools

import jax
import jax.numpy as jnp
from jax import lax
from jax.experimental import pallas as pl
from jax.experimental.pallas import tpu as pltpu
from jax.experimental.pallas import tpu_sc as plsc

_NC = 2
_NS = 16
_LANES = 16


def _s1_body(fea_ref, wl_ref, bl_ref, x_ref, s_ref, ss_ref):
    i = pl.program_id(0)
    x = jnp.dot(fea_ref[...], wl_ref[...], preferred_element_type=jnp.float32)
    x = x + bl_ref[...]
    x_ref[...] = x

    @pl.when(i == 0)
    def _():
        s_ref[...] = jnp.zeros_like(s_ref)
        ss_ref[...] = jnp.zeros_like(ss_ref)

    s_ref[...] += jnp.sum(x, axis=0, keepdims=True)
    ss_ref[...] += jnp.sum(x * x, axis=0, keepdims=True)


def _make_s2(n_rows, d_half):
    def body(x_ref, s_ref, ss_ref, g_ref, b_ref, wg_ref, d0_ref, d1_ref,
             xz_ref):
        inv_n = 1.0 / n_rows
        mean = s_ref[...] * inv_n
        var = ss_ref[...] * inv_n - mean * mean
        rstd = lax.rsqrt(var + 1e-5)
        xn = (x_ref[...] - mean) * rstd * g_ref[...] + b_ref[...]
        xr = jnp.where(xn >= 0, xn, 0.01 * xn)
        xw = jnp.dot(xr, wg_ref[...], preferred_element_type=jnp.float32)
        deg = d0_ref[...] + d1_ref[...] + 1.0
        dinv = lax.rsqrt(deg)
        xz = xw * dinv
        xz_ref[0] = xz[:, :d_half]
        xz_ref[1] = xz[:, d_half:]
    return body


def _make_s4(d_half):
    def body(al_ref, ar_ref, xl_ref, xr_ref, d0_ref, d1_ref, bg_ref, out_ref):
        deg = d0_ref[...] + d1_ref[...] + 1.0
        dinv = lax.rsqrt(deg)
        out_ref[:, :d_half] = dinv * (al_ref[...] + xl_ref[...]) + bg_ref[:, :d_half]
        out_ref[:, d_half:] = dinv * (ar_ref[...] + xr_ref[...]) + bg_ref[:, d_half:]
    return body


def _make_deg(np_pad, ep32, ch32):
    rt = np_pad // _NS

    def body(cols3, w_hbm, degp, cbuf, wbuf, dbuf, deg_sp):
        c = lax.axis_index("c")
        s = lax.axis_index("s")
        t = c * _NS + s

        def zstep(i, carry):
            dbuf[pl.ds(i * _LANES, _LANES)] = jnp.zeros((_LANES,), jnp.float32)
            return carry

        lax.fori_loop(0, rt // _LANES, zstep, 0)
        pltpu.sync_copy(dbuf, deg_sp.at[pl.ds(s * rt, rt)])
        pltpu.sync_copy(cols3.at[t], cbuf)
        pltpu.sync_copy(w_hbm.at[pl.ds(t * ep32, ep32)], wbuf)
        plsc.subcore_barrier()

        def chunk(g, carry):
            pltpu.sync_copy(wbuf.at[pl.ds(g * 128, 128)],
                            deg_sp.at[cbuf.at[g]], add=True)
            return carry

        lax.fori_loop(0, ch32, chunk, 0)
        plsc.subcore_barrier()
        pltpu.sync_copy(deg_sp.at[pl.ds(s * rt, rt)], dbuf)
        pltpu.sync_copy(dbuf, degp.at[pl.ds(c * np_pad + s * rt, rt)])
    return body


_CK = 64
_NBUF = 4


def _make_agg(np_pad, d_half, ep16, ch16):
    rt = np_pad // _NS
    nvec = d_half // _LANES

    def body(rows2, cols_hbm, w_hbm, xzflat, agg3,
             ibufs, cbufs, vbufs, gbufs, acc, sems_g, sems_s, sems_i,
             sems_c):
        c = lax.axis_index("c")
        s = lax.axis_index("s")
        ebase = s * ep16

        def zstep(i, carry):
            for j in range(nvec):
                gbufs[0][i, pl.ds(j * _LANES, _LANES)] = (
                    jnp.zeros((_LANES,), jnp.float32))
            return carry

        lax.fori_loop(0, _CK, zstep, 0)
        for k in range(rt // _CK):
            pltpu.sync_copy(gbufs[0], acc.at[pl.ds(s * rt + k * _CK, _CK)])
        plsc.subcore_barrier()

        def idx_load(g, b):
            pltpu.async_copy(rows2.at[c, pl.ds(ebase + g * _CK, _CK)],
                             ibufs[b], sems_i[b])
            pltpu.async_copy(w_hbm.at[pl.ds(ebase + g * _CK, _CK)],
                             vbufs[b], sems_i[b])

        def idx_wait(g, b):
            pltpu.make_async_copy(rows2.at[c, pl.ds(ebase + g * _CK, _CK)],
                                  ibufs[b], sems_i[b]).wait()
            pltpu.make_async_copy(w_hbm.at[pl.ds(ebase + g * _CK, _CK)],
                                  vbufs[b], sems_i[b]).wait()

        def cols_load(g, b):
            pltpu.async_copy(cols_hbm.at[pl.ds(ebase + g * _CK, _CK)],
                             cbufs[b], sems_c[b])

        def cols_wait(g, b):
            pltpu.make_async_copy(cols_hbm.at[pl.ds(ebase + g * _CK, _CK)],
                                  cbufs[b], sems_c[b]).wait()

        def gather(b):
            pltpu.async_copy(xzflat.at[ibufs[b]], gbufs[b], sems_g[b])

        def gather_wait(b):
            pltpu.make_async_copy(xzflat.at[ibufs[b]], gbufs[b],
                                  sems_g[b]).wait()

        def scale(b):
            gb = gbufs[b]
            vb = vbufs[b]

            @plsc.parallel_loop(0, _CK, step=1, unroll=4)
            def _(e):
                idx = jnp.broadcast_to(e, (_LANES,)).astype(jnp.int32)
                sp = plsc.load_gather(vb, [idx])
                for j in range(nvec):
                    sl = pl.ds(j * _LANES, _LANES)
                    gb[e, sl] = gb[e, sl] * sp

        def scatter(b):
            pltpu.async_copy(gbufs[b], acc.at[cbufs[b]], sems_s[b], add=True)

        def scatter_wait(b):
            pltpu.make_async_copy(gbufs[b], acc.at[cbufs[b]],
                                  sems_s[b]).wait()

        for q in range(_NBUF):
            idx_load(q, q)
            cols_load(q, q)
        for q in range(_NBUF - 1):
            idx_wait(q, q)
            gather(q)

        def quad(p, carry):
            for q in range(_NBUF):
                g = p * _NBUF + q
                b = q
                bp = (q - 1) % _NBUF
                gather_wait(b)
                scale(b)

                @pl.when(g > 0)
                def _():
                    scatter_wait(bp)

                    @pl.when(g + _NBUF - 1 < ch16)
                    def _():
                        cols_load(g + _NBUF - 1, bp)

                @pl.when(g + _NBUF - 1 < ch16)
                def _():
                    idx_wait(g + _NBUF - 1, bp)
                    gather(bp)

                @pl.when(g + _NBUF < ch16)
                def _():
                    idx_load(g + _NBUF, b)

                cols_wait(g, b)
                scatter(b)
            return carry

        lax.fori_loop(0, ch16 // _NBUF, quad, 0)
        scatter_wait(_NBUF - 1)
        plsc.subcore_barrier()
        for k in range(rt // _CK):
            pltpu.sync_copy(acc.at[pl.ds(s * rt + k * _CK, _CK)], gbufs[0])
            pltpu.sync_copy(
                gbufs[0], agg3.at[pl.ds(c * np_pad + s * rt + k * _CK, _CK)])
    return body


def kernel(fea, edges, weights, W_lin, b_lin, gamma, beta, W_gcn, b_gcn):
    N, D = fea.shape
    DH = D // 2
    E = edges.shape[1]

    EPAD = -(-E // (_NC * _NS * 128)) * (_NC * _NS * 128)
    EP16 = EPAD // _NS
    CH16 = EP16 // _CK
    EP32 = EPAD // (_NC * _NS)
    CH32 = EP32 // 128
    NP = -(-N // 2048) * 2048
    R = 1000
    G = N // R

    rows = edges[0].astype(jnp.int32)
    cols = edges[1].astype(jnp.int32)
    pad = EPAD - E
    rows_p = jnp.concatenate([rows, jnp.zeros((pad,), jnp.int32)])
    cols_p = jnp.concatenate([cols, jnp.zeros((pad,), jnp.int32)])
    w_p = jnp.concatenate([weights.astype(jnp.float32),
                           jnp.zeros((pad,), jnp.float32)])
    rows2 = jnp.stack([rows_p, rows_p + N])
    cols3_32 = cols_p.reshape(_NC * _NS, CH32, 128)

    X, ssum, ssq = pl.pallas_call(
        _s1_body,
        grid=(G,),
        in_specs=[pl.BlockSpec((R, D), lambda i: (i, 0)),
                  pl.BlockSpec((D, D), lambda i: (0, 0)),
                  pl.BlockSpec((1, D), lambda i: (0, 0))],
        out_specs=[pl.BlockSpec((R, D), lambda i: (i, 0)),
                   pl.BlockSpec((1, D), lambda i: (0, 0)),
                   pl.BlockSpec((1, D), lambda i: (0, 0))],
        out_shape=[jax.ShapeDtypeStruct((N, D), jnp.float32),
                   jax.ShapeDtypeStruct((1, D), jnp.float32),
                   jax.ShapeDtypeStruct((1, D), jnp.float32)],
    )(fea, W_lin, b_lin.reshape(1, D))

    mesh = plsc.VectorSubcoreMesh(core_axis_name="c", subcore_axis_name="s")
    sc_params = pltpu.CompilerParams(needs_layout_passes=False)
    degp = pl.kernel(
        _make_deg(NP, EP32, CH32),
        out_type=jax.ShapeDtypeStruct((_NC * NP,), jnp.float32),
        mesh=mesh,
        compiler_params=sc_params,
        scratch_types=[
            pltpu.VMEM((CH32, 128), jnp.int32),
            pltpu.VMEM((EP32,), jnp.float32),
            pltpu.VMEM((NP // _NS,), jnp.float32),
            pltpu.VMEM_SHARED((NP,), jnp.float32),
        ],
    )(cols3_32, w_p)

    deg0 = degp[:N].reshape(N, 1)
    deg1 = degp[NP:NP + N].reshape(N, 1)

    xz3 = pl.pallas_call(
        _make_s2(N, DH),
        grid=(G,),
        in_specs=[pl.BlockSpec((R, D), lambda i: (i, 0)),
                  pl.BlockSpec((1, D), lambda i: (0, 0)),
                  pl.BlockSpec((1, D), lambda i: (0, 0)),
                  pl.BlockSpec((1, D), lambda i: (0, 0)),
                  pl.BlockSpec((1, D), lambda i: (0, 0)),
                  pl.BlockSpec((D, D), lambda i: (0, 0)),
                  pl.BlockSpec((R, 1), lambda i: (i, 0)),
                  pl.BlockSpec((R, 1), lambda i: (i, 0))],
        out_specs=pl.BlockSpec((_NC, R, DH), lambda i: (0, i, 0)),
        out_shape=jax.ShapeDtypeStruct((_NC, N, DH), jnp.float32),
    )(X, ssum, ssq, gamma.reshape(1, D), beta.reshape(1, D), W_gcn,
      deg0, deg1)

    xzflat = xz3.reshape(_NC * N, DH)

    agg2 = pl.kernel(
        _make_agg(NP, DH, EP16, CH16),
        out_type=jax.ShapeDtypeStruct((_NC * NP, DH), jnp.float32),
        mesh=mesh,
        compiler_params=sc_params,
        scratch_types=[
            [pltpu.VMEM((_CK,), jnp.int32) for _ in range(_NBUF)],
            [pltpu.VMEM((_CK,), jnp.int32) for _ in range(_NBUF)],
            [pltpu.VMEM((_CK,), jnp.float32) for _ in range(_NBUF)],
            [pltpu.VMEM((_CK, DH), jnp.float32) for _ in range(_NBUF)],
            pltpu.VMEM_SHARED((NP, DH), jnp.float32),
            [pltpu.SemaphoreType.DMA for _ in range(_NBUF)],
            [pltpu.SemaphoreType.DMA for _ in range(_NBUF)],
            [pltpu.SemaphoreType.DMA for _ in range(_NBUF)],
            [pltpu.SemaphoreType.DMA for _ in range(_NBUF)],
        ],
    )(rows2, cols_p, w_p, xzflat)
    agg_l = agg2[:N]
    agg_r = agg2[NP:NP + N]
    xz_l = xz3[0]
    xz_r = xz3[1]

    out = pl.pallas_call(
        _make_s4(DH),
        grid=(G,),
        in_specs=[pl.BlockSpec((R, DH), lambda i: (i, 0)),
                  pl.BlockSpec((R, DH), lambda i: (i, 0)),
                  pl.BlockSpec((R, DH), lambda i: (i, 0)),
                  pl.BlockSpec((R, DH), lambda i: (i, 0)),
                  pl.BlockSpec((R, 1), lambda i: (i, 0)),
                  pl.BlockSpec((R, 1), lambda i: (i, 0)),
                  pl.BlockSpec((1, D), lambda i: (0, 0))],
        out_specs=pl.BlockSpec((R, D), lambda i: (i, 0)),
        out_shape=jax.ShapeDtypeStruct((N, D), jnp.float32),
    )(agg_l, agg_r, xz_l, xz_r, deg0, deg1, b_gcn.reshape(1, D))
    return out
